# Initial kernel scaffold; baseline (speedup 1.0000x reference)
import jax
import jax.numpy as jnp
from jax.experimental import pallas as pl


def kernel(x, batch, W1, b1, W2, b2, W3, b3):
    raise NotImplementedError("write your pallas kernel here")


# speedup vs baseline: 10.4513x; 10.4513x over previous
import functools

import jax
import jax.numpy as jnp
from jax import lax
from jax.experimental import pallas as pl
from jax.experimental.pallas import tpu as pltpu
from jax.experimental.pallas import tpu_sc as plsc

N = 320000
D = 128
H = 128
NUM_SEG = 10000


BN = 2000


def _mlp_body(x_ref, w1_ref, b1_ref, w2_ref, b2_ref, w3_ref, b3_ref, o_ref):
    h = jnp.dot(x_ref[...], w1_ref[...], preferred_element_type=jnp.float32)
    h = jnp.maximum(h + b1_ref[...], 0.0)
    h = jnp.dot(h, w2_ref[...], preferred_element_type=jnp.float32)
    h = jnp.maximum(h + b2_ref[...], 0.0)
    s = jnp.dot(h, w3_ref[...], preferred_element_type=jnp.float32)
    o_ref[...] = s + b3_ref[...]


def _mlp_scalar(x, W1, b1, W2, b2, W3, b3):
    grid = N // BN
    full = lambda i: (0, 0)
    out = pl.pallas_call(
        _mlp_body,
        grid=(grid,),
        in_specs=[
            pl.BlockSpec((BN, D), lambda i: (i, 0)),
            pl.BlockSpec((D, H), full),
            pl.BlockSpec((1, H), full),
            pl.BlockSpec((H, H), full),
            pl.BlockSpec((1, H), full),
            pl.BlockSpec((H, 1), full),
            pl.BlockSpec((1, 1), full),
        ],
        out_specs=pl.BlockSpec((BN, 1), lambda i: (i, 0)),
        out_shape=jax.ShapeDtypeStruct((N, 1), jnp.float32),
    )(x, W1, b1.reshape(1, H), W2, b2.reshape(1, H), W3, b3.reshape(1, 1))
    return out.reshape(N)


NS = 16
L = 16
CHUNK = N // NS
TBL = 10240
SEG = TBL // NS


def _sc_body(s_hbm, batch_hbm, out_hbm,
             ids_v, s_v, pooled_v, counts_v, accp_v, accc_v, tmp_v, wfull_v,
             shared_pooled, shared_counts, shared_w):
    sid = lax.axis_index("s")
    base = sid * CHUNK

    pltpu.sync_copy(batch_hbm.at[pl.ds(base, CHUNK)], ids_v)
    pltpu.sync_copy(s_hbm.at[pl.ds(base, CHUNK)], s_v)

    zeros = jnp.zeros((L,), jnp.float32)

    def zinit(i, _):
        pooled_v[pl.ds(i * L, L)] = zeros
        counts_v[pl.ds(i * L, L)] = zeros
        return _

    lax.fori_loop(0, TBL // L, zinit, 0)

    ones = jnp.ones((L,), jnp.float32)

    def accum(i, _):
        ids = ids_v[pl.ds(i * L, L)]
        sv = s_v[pl.ds(i * L, L)]
        plsc.addupdate_scatter(pooled_v, [ids], sv)
        plsc.addupdate_scatter(counts_v, [ids], ones)
        return _

    lax.fori_loop(0, CHUNK // L, accum, 0)

    pltpu.sync_copy(pooled_v, shared_pooled.at[sid])
    pltpu.sync_copy(counts_v, shared_counts.at[sid])
    plsc.subcore_barrier()

    segbase = sid * SEG
    pltpu.sync_copy(shared_pooled.at[0, pl.ds(segbase, SEG)], accp_v)
    pltpu.sync_copy(shared_counts.at[0, pl.ds(segbase, SEG)], accc_v)

    def addp(k, _):
        accp_v[pl.ds(k * L, L)] = accp_v[pl.ds(k * L, L)] + tmp_v[pl.ds(k * L, L)]
        return _

    def addc(k, _):
        accc_v[pl.ds(k * L, L)] = accc_v[pl.ds(k * L, L)] + tmp_v[pl.ds(k * L, L)]
        return _

    for r in range(1, NS):
        pltpu.sync_copy(shared_pooled.at[r, pl.ds(segbase, SEG)], tmp_v)
        lax.fori_loop(0, SEG // L, addp, 0)
        pltpu.sync_copy(shared_counts.at[r, pl.ds(segbase, SEG)], tmp_v)
        lax.fori_loop(0, SEG // L, addc, 0)

    def wcomp(k, _):
        p = accp_v[pl.ds(k * L, L)]
        c = accc_v[pl.ds(k * L, L)]
        accp_v[pl.ds(k * L, L)] = jnp.maximum(p / c, 0.0)
        return _

    lax.fori_loop(0, SEG // L, wcomp, 0)

    pltpu.sync_copy(accp_v, shared_w.at[pl.ds(segbase, SEG)])
    plsc.subcore_barrier()
    pltpu.sync_copy(shared_w, wfull_v)

    def gath(i, _):
        ids = ids_v[pl.ds(i * L, L)]
        s_v[pl.ds(i * L, L)] = plsc.load_gather(wfull_v, [ids])
        return _

    lax.fori_loop(0, CHUNK // L, gath, 0)

    pltpu.sync_copy(s_v, out_hbm.at[pl.ds(base, CHUNK)])


@functools.partial(jax.jit, static_argnames=())
def _sc_segment(s, batch):
    mesh = plsc.VectorSubcoreMesh(
        core_axis_name="c", subcore_axis_name="s", num_cores=1
    )
    return pl.kernel(
        _sc_body,
        out_type=jax.ShapeDtypeStruct((N,), jnp.float32),
        mesh=mesh,
        compiler_params=pltpu.CompilerParams(needs_layout_passes=False),
        scratch_types=[
            pltpu.VMEM((CHUNK,), jnp.int32),
            pltpu.VMEM((CHUNK,), jnp.float32),
            pltpu.VMEM((TBL,), jnp.float32),
            pltpu.VMEM((TBL,), jnp.float32),
            pltpu.VMEM((SEG,), jnp.float32),
            pltpu.VMEM((SEG,), jnp.float32),
            pltpu.VMEM((SEG,), jnp.float32),
            pltpu.VMEM((TBL,), jnp.float32),
            pltpu.VMEM_SHARED((NS, TBL), jnp.float32),
            pltpu.VMEM_SHARED((NS, TBL), jnp.float32),
            pltpu.VMEM_SHARED((TBL,), jnp.float32),
        ],
    )(s, batch)


def kernel(x, batch, W1, b1, W2, b2, W3, b3):
    batch = batch.astype(jnp.int32)
    x = x.astype(jnp.float32)
    s = _mlp_scalar(x, W1, b1, W2, b2, W3, b3)
    return _sc_segment(s, batch)
